# baseline (speedup 1.0000x reference)
import functools

import jax
import jax.numpy as jnp
from jax import lax
from jax.experimental import pallas as pl
from jax.experimental.pallas import tpu as pltpu
from jax.experimental.pallas import tpu_sc as plsc

_V = 100000
_F = 100000
_E = 3200000
_ALPHA = 0.4
_MAX_COEFF = 10.0

_NC = 2
_NS = 16
_NW = _NC * _NS
_LANES = 16

_C = 800
_EPW = _E // _NW
_NFULL = _EPW // _C
_NPAIR = (_NFULL - 1) // 2

_F_PAD = 100352
_SLICE = _F_PAD // _NS
_ZB = 2 * _SLICE // 8
_ROWS = _F_PAD // 128


def _sc_body(vp_hbm, vidx_hbm, fidx_hbm, ef_hbm, coeff_hbm,
             nd_out,
             vp_v, vidx0, vidx1, fidx0, fidx1, ef0, ef1,
             nd0, nd1, fsc0, fsc1, coeff_v, zbuf,
             spnd, sin0, sin1, ssc0, ssc1):
    cid = lax.axis_index("c")
    sid = lax.axis_index("s")
    wid = cid * _NS + sid
    base = wid * _EPW

    vin = (vidx0, vidx1)
    fin = (fidx0, fidx1)
    ein = (ef0, ef1)
    nd = (nd0, nd1)
    fsc = (fsc0, fsc1)
    sin = (sin0, sin1)
    ssc = (ssc0, ssc1)

    lane2 = jax.lax.iota(jnp.int32, _LANES) * 2

    tbl = pltpu.async_copy(vp_hbm, vp_v, ssc0)
    pltpu.sync_copy(coeff_hbm, coeff_v)

    def _zero(i, _):
        zbuf[pl.ds(i * _LANES, _LANES)] = jnp.zeros((_LANES,), jnp.float32)
        return 0
    lax.fori_loop(0, _ZB // _LANES, _zero, 0)
    for t in range(8):
        dst = pl.ds(2 * sid * _SLICE + t * _ZB, _ZB)
        pltpu.sync_copy(zbuf, spnd.at[dst])
    tbl.wait()
    plsc.subcore_barrier()

    coeff = coeff_v[...]

    def _in_descs(off, b):
        return (
            pltpu.make_async_copy(vidx_hbm.at[pl.ds(off, _C)], vin[b], sin[b]),
            pltpu.make_async_copy(fidx_hbm.at[pl.ds(off, _C)], fin[b], sin[b]),
            pltpu.make_async_copy(ef_hbm.at[pl.ds(off, _C)], ein[b], sin[b]),
        )

    def _sc_desc(b):
        return pltpu.make_async_copy(nd[b], spnd.at[fsc[b]], ssc[b])

    def _compute(b):
        def _vec(j, _):
            s = pl.ds(j * _LANES, _LANES)
            idx = vin[b][s]
            f = fin[b][s]
            e = ein[b][s]
            v = plsc.load_gather(vp_v, [idx])
            ev = e * v + (0.5 - 0.5 * e)
            w = jnp.exp(coeff * ev)
            f2 = f * 2
            rows = lane2 + (2 * _LANES) * j
            plsc.store_scatter(nd[b], [rows], w * ev)
            plsc.store_scatter(nd[b], [rows + 1], w)
            plsc.store_scatter(fsc[b], [rows], f2)
            plsc.store_scatter(fsc[b], [rows + 1], f2 + 1)
            return 0
        lax.fori_loop(0, _C // _LANES, _vec, 0)

    for d in _in_descs(pl.multiple_of(base, _C), 0):
        d.start()

    def _pair(k, _):
        for b in (0, 1):
            i = 2 * k + b
            off = pl.multiple_of(base + i * _C, _C)
            for d in _in_descs(off + _C, 1 - b):
                d.start()
            @pl.when(k >= 1)
            def _():
                _sc_desc(b).wait()
            for d in _in_descs(off, b):
                d.wait()
            _compute(b)
            pltpu.async_copy(nd[b], spnd.at[fsc[b]], ssc[b], add=True)
        return 0
    lax.fori_loop(0, _NPAIR, _pair, 0)

    off = pl.multiple_of(base + 2 * _NPAIR * _C, _C)
    _sc_desc(0).wait()
    for d in _in_descs(off, 0):
        d.wait()
    _compute(0)
    pltpu.async_copy(nd[0], spnd.at[fsc[0]], ssc[0], add=True)

    _sc_desc(1).wait()
    _sc_desc(0).wait()

    plsc.subcore_barrier()

    src = pl.ds(2 * sid * _SLICE, 2 * _SLICE)
    dst = pl.ds(2 * (cid * _F_PAD + sid * _SLICE), 2 * _SLICE)
    pltpu.sync_copy(spnd.at[src], nd_out.at[dst])


@functools.lru_cache(maxsize=None)
def _build_sc_segment():
  return pl.kernel(
    _sc_body,
    out_type=jax.ShapeDtypeStruct((_NC * 2 * _F_PAD,), jnp.float32),
    mesh=plsc.VectorSubcoreMesh(
        core_axis_name="c", subcore_axis_name="s",
        num_cores=_NC, num_subcores=_NS),
    compiler_params=pltpu.CompilerParams(needs_layout_passes=False),
    scratch_types=(
        pltpu.VMEM((_V,), jnp.float32),
        pltpu.VMEM((_C,), jnp.int32),
        pltpu.VMEM((_C,), jnp.int32),
        pltpu.VMEM((_C,), jnp.int32),
        pltpu.VMEM((_C,), jnp.int32),
        pltpu.VMEM((_C,), jnp.float32),
        pltpu.VMEM((_C,), jnp.float32),
        pltpu.VMEM((2 * _C,), jnp.float32),
        pltpu.VMEM((2 * _C,), jnp.float32),
        pltpu.VMEM((2 * _C,), jnp.int32),
        pltpu.VMEM((2 * _C,), jnp.int32),
        pltpu.VMEM((_LANES,), jnp.float32),
        pltpu.VMEM((_ZB,), jnp.float32),
        pltpu.VMEM_SHARED((2 * _F_PAD,), jnp.float32),
        pltpu.SemaphoreType.DMA,
        pltpu.SemaphoreType.DMA,
        pltpu.SemaphoreType.DMA,
        pltpu.SemaphoreType.DMA,
    ),
  )


def _tc_loss_body(eps_ref, nom_ref, den_ref, out_ref):
    eps = eps_ref[0, 0]
    nom = nom_ref[0] + nom_ref[1]
    den = den_ref[0] + den_ref[1]
    cv = den / jnp.maximum(nom, eps)
    t = cv - 1.0
    t2 = t * t
    cv5 = 1.0 + t2 * t2 * t
    lg = jnp.log(jnp.maximum(cv5, eps))
    rows = lax.broadcasted_iota(jnp.int32, (_ROWS, 128), 0)
    cols = lax.broadcasted_iota(jnp.int32, (_ROWS, 128), 1)
    valid = (rows * 128 + cols) < _F
    out_ref[0, 0] = jnp.sum(jnp.where(valid, lg, 0.0)) * (1.0 / _F)


_tc_loss = pl.pallas_call(
    _tc_loss_body,
    out_shape=jax.ShapeDtypeStruct((1, 1), jnp.float32),
    in_specs=[
        pl.BlockSpec(memory_space=pltpu.SMEM),
        pl.BlockSpec(memory_space=pltpu.VMEM),
        pl.BlockSpec(memory_space=pltpu.VMEM),
    ],
    out_specs=pl.BlockSpec(memory_space=pltpu.SMEM),
)


def kernel(variable_prediction, label, graph_map, batch_variable_map,
           batch_function_map, edge_feature, meta_data, global_step, eps):
    coeff = jnp.minimum(jnp.power(global_step, _ALPHA),
                        jnp.float32(_MAX_COEFF))
    coeff16 = jnp.broadcast_to(coeff.astype(jnp.float32), (_LANES,))
    vp = variable_prediction.reshape(_V)
    ef = edge_feature.reshape(_E)
    vidx = graph_map[0]
    fidx = graph_map[1]
    nd_flat = _build_sc_segment()(vp, vidx, fidx, ef, coeff16)
    nd = nd_flat.reshape(_NC, _F_PAD, 2)
    nom3 = nd[..., 0].reshape(_NC, _ROWS, 128)
    den3 = nd[..., 1].reshape(_NC, _ROWS, 128)
    loss = _tc_loss(eps.reshape(1, 1), nom3, den3)
    return loss[0, 0]


# speedup vs baseline: 2.3399x; 2.3399x over previous
import functools

import jax
import jax.numpy as jnp
from jax import lax
from jax.experimental import pallas as pl
from jax.experimental.pallas import tpu as pltpu
from jax.experimental.pallas import tpu_sc as plsc

_V = 100000
_F = 100000
_E = 3200000
_ALPHA = 0.4
_MAX_COEFF = 10.0

_NC = 2
_NS = 16
_NW = _NC * _NS
_LANES = 16

_C = 800
_EPW = _E // _NW
_NFULL = _EPW // _C
_NPAIR = (_NFULL - 1) // 2

_F_PAD = 100352
_SLICE = _F_PAD // _NS
_ZB = _SLICE // 2
_ROWS = _F_PAD // 128


def _sc_body(vp_hbm, vidx_hbm, fidx_hbm, ef_hbm, coeff_hbm,
             nom_out, den_out,
             vp_v, vidx0, vidx1, fidx0, fidx1, ef0, ef1,
             nomv0, nomv1, denv0, denv1, fsc0, fsc1, coeff_v, zbuf,
             spn, spd, sin0, sin1, ssn0, ssn1, ssd0, ssd1):
    cid = lax.axis_index("c")
    sid = lax.axis_index("s")
    wid = cid * _NS + sid
    base = wid * _EPW

    vin = (vidx0, vidx1)
    fin = (fidx0, fidx1)
    ein = (ef0, ef1)
    nomv = (nomv0, nomv1)
    denv = (denv0, denv1)
    fsc = (fsc0, fsc1)
    sin = (sin0, sin1)
    ssn = (ssn0, ssn1)
    ssd = (ssd0, ssd1)

    tbl = pltpu.async_copy(vp_hbm, vp_v, ssn0)
    pltpu.sync_copy(coeff_hbm, coeff_v)

    def _zero(i, _):
        zbuf[pl.ds(i * _LANES, _LANES)] = jnp.zeros((_LANES,), jnp.float32)
        return 0
    lax.fori_loop(0, _ZB // _LANES, _zero, 0)
    for t in range(2):
        dst = pl.ds(sid * _SLICE + t * _ZB, _ZB)
        pltpu.sync_copy(zbuf, spn.at[dst])
        pltpu.sync_copy(zbuf, spd.at[dst])
    tbl.wait()
    plsc.subcore_barrier()

    coeff = coeff_v[...]

    def _in_descs(off, b):
        return (
            pltpu.make_async_copy(vidx_hbm.at[pl.ds(off, _C)], vin[b], sin[b]),
            pltpu.make_async_copy(fidx_hbm.at[pl.ds(off, _C)], fin[b], sin[b]),
            pltpu.make_async_copy(ef_hbm.at[pl.ds(off, _C)], ein[b], sin[b]),
        )

    def _sc_descs(b):
        return (
            pltpu.make_async_copy(nomv[b], spn.at[fsc[b]], ssn[b]),
            pltpu.make_async_copy(denv[b], spd.at[fsc[b]], ssd[b]),
        )

    def _compute(b):
        def _one(j):
            s = pl.ds(j * _LANES, _LANES)
            idx = vin[b][s]
            f = fin[b][s]
            e = ein[b][s]
            v = plsc.load_gather(vp_v, [idx])
            ev = e * v + (0.5 - 0.5 * e)
            w = jnp.exp(coeff * ev)
            nomv[b][s] = w * ev
            denv[b][s] = w
            fsc[b][s] = f

        def _vec(j, _):
            _one(2 * j)
            _one(2 * j + 1)
            return 0
        lax.fori_loop(0, _C // (2 * _LANES), _vec, 0)

    for d in _in_descs(pl.multiple_of(base, _C), 0):
        d.start()

    def _pair(k, _):
        for b in (0, 1):
            i = 2 * k + b
            off = pl.multiple_of(base + i * _C, _C)
            for d in _in_descs(off + _C, 1 - b):
                d.start()
            @pl.when(k >= 1)
            def _():
                for d in _sc_descs(b):
                    d.wait()
            for d in _in_descs(off, b):
                d.wait()
            _compute(b)
            pltpu.async_copy(nomv[b], spn.at[fsc[b]], ssn[b], add=True)
            pltpu.async_copy(denv[b], spd.at[fsc[b]], ssd[b], add=True)
        return 0
    lax.fori_loop(0, _NPAIR, _pair, 0)

    off = pl.multiple_of(base + 2 * _NPAIR * _C, _C)
    for d in _sc_descs(0):
        d.wait()
    for d in _in_descs(off, 0):
        d.wait()
    _compute(0)
    pltpu.async_copy(nomv[0], spn.at[fsc[0]], ssn[0], add=True)
    pltpu.async_copy(denv[0], spd.at[fsc[0]], ssd[0], add=True)

    for b in (1, 0):
        for d in _sc_descs(b):
            d.wait()

    plsc.subcore_barrier()

    src = pl.ds(sid * _SLICE, _SLICE)
    dst = pl.ds(cid * _F_PAD + sid * _SLICE, _SLICE)
    pltpu.sync_copy(spn.at[src], nom_out.at[dst])
    pltpu.sync_copy(spd.at[src], den_out.at[dst])


@functools.lru_cache(maxsize=None)
def _build_sc_segment():
  return pl.kernel(
    _sc_body,
    out_type=(
        jax.ShapeDtypeStruct((_NC * _F_PAD,), jnp.float32),
        jax.ShapeDtypeStruct((_NC * _F_PAD,), jnp.float32),
    ),
    mesh=plsc.VectorSubcoreMesh(
        core_axis_name="c", subcore_axis_name="s",
        num_cores=_NC, num_subcores=_NS),
    compiler_params=pltpu.CompilerParams(needs_layout_passes=False),
    scratch_types=(
        pltpu.VMEM((_V,), jnp.float32),
        pltpu.VMEM((_C,), jnp.int32),
        pltpu.VMEM((_C,), jnp.int32),
        pltpu.VMEM((_C,), jnp.int32),
        pltpu.VMEM((_C,), jnp.int32),
        pltpu.VMEM((_C,), jnp.float32),
        pltpu.VMEM((_C,), jnp.float32),
        pltpu.VMEM((_C,), jnp.float32),
        pltpu.VMEM((_C,), jnp.float32),
        pltpu.VMEM((_C,), jnp.float32),
        pltpu.VMEM((_C,), jnp.float32),
        pltpu.VMEM((_C,), jnp.int32),
        pltpu.VMEM((_C,), jnp.int32),
        pltpu.VMEM((_LANES,), jnp.float32),
        pltpu.VMEM((_ZB,), jnp.float32),
        pltpu.VMEM_SHARED((_F_PAD,), jnp.float32),
        pltpu.VMEM_SHARED((_F_PAD,), jnp.float32),
        pltpu.SemaphoreType.DMA,
        pltpu.SemaphoreType.DMA,
        pltpu.SemaphoreType.DMA,
        pltpu.SemaphoreType.DMA,
        pltpu.SemaphoreType.DMA,
        pltpu.SemaphoreType.DMA,
    ),
  )


def _tc_loss_body(eps_ref, nom_ref, den_ref, out_ref):
    eps = eps_ref[0, 0]
    nom = nom_ref[0] + nom_ref[1]
    den = den_ref[0] + den_ref[1]
    cv = den / jnp.maximum(nom, eps)
    t = cv - 1.0
    t2 = t * t
    cv5 = 1.0 + t2 * t2 * t
    lg = jnp.log(jnp.maximum(cv5, eps))
    rows = lax.broadcasted_iota(jnp.int32, (_ROWS, 128), 0)
    cols = lax.broadcasted_iota(jnp.int32, (_ROWS, 128), 1)
    valid = (rows * 128 + cols) < _F
    out_ref[0, 0] = jnp.sum(jnp.where(valid, lg, 0.0)) * (1.0 / _F)


_tc_loss = pl.pallas_call(
    _tc_loss_body,
    out_shape=jax.ShapeDtypeStruct((1, 1), jnp.float32),
    in_specs=[
        pl.BlockSpec(memory_space=pltpu.SMEM),
        pl.BlockSpec(memory_space=pltpu.VMEM),
        pl.BlockSpec(memory_space=pltpu.VMEM),
    ],
    out_specs=pl.BlockSpec(memory_space=pltpu.SMEM),
)


def kernel(variable_prediction, label, graph_map, batch_variable_map,
           batch_function_map, edge_feature, meta_data, global_step, eps):
    coeff = jnp.minimum(jnp.power(global_step, _ALPHA),
                        jnp.float32(_MAX_COEFF))
    coeff16 = jnp.broadcast_to(coeff.astype(jnp.float32), (_LANES,))
    vp = variable_prediction.reshape(_V)
    ef = edge_feature.reshape(_E)
    vidx = graph_map[0]
    fidx = graph_map[1]
    nom_flat, den_flat = _build_sc_segment()(vp, vidx, fidx, ef, coeff16)
    nom3 = nom_flat.reshape(_NC, _ROWS, 128)
    den3 = den_flat.reshape(_NC, _ROWS, 128)
    loss = _tc_loss(eps.reshape(1, 1), nom3, den3)
    return loss[0, 0]
